# Initial kernel scaffold; baseline (speedup 1.0000x reference)
import jax
import jax.numpy as jnp
from jax.experimental import pallas as pl


def kernel(queries, keys, values):
    raise NotImplementedError("write your pallas kernel here")


# speedup vs baseline: 2.4709x; 2.4709x over previous
import functools
import math

import numpy as np
import jax
import jax.numpy as jnp
from jax.experimental import pallas as pl
from jax.experimental.pallas import tpu as pltpu

_FACTOR = 5
_NEG = np.float32(-3.4e38)


@functools.lru_cache(maxsize=None)
def _sample_count_matrix_T(L_Q: int, L_K: int, U_part: int) -> np.ndarray:
    with jax.ensure_compile_time_eval():
        idx = jax.random.randint(jax.random.key(42), (L_Q, U_part), 0, L_K)
    idx_np = np.asarray(idx, dtype=np.int64)
    ct = np.zeros((L_K, L_Q), np.float32)
    np.add.at(ct, (idx_np.reshape(-1), np.repeat(np.arange(L_Q), U_part)), 1.0)
    return ct


def _body(u, u_pad, L_K, scale, q_ref, k_ref, v_ref, ct_ref, o_ref):
    L_Q, D = q_ref.shape[1], q_ref.shape[2]
    Q = q_ref[0]
    K = k_ref[0]
    V = v_ref[0]

    CH = min(512, L_Q)
    n_ch = L_Q // CH
    rows = []
    for ci in range(n_ch):
        qc = Q[ci * CH:(ci + 1) * CH, :]
        qkt = jax.lax.dot_general(K, qc, (((1,), (1,)), ((), ())),
                                  preferred_element_type=jnp.float32)
        cct = ct_ref[:, ci * CH:(ci + 1) * CH]
        smax = jnp.max(jnp.where(cct > 0, qkt, _NEG), axis=0)
        ssum = jnp.sum(qkt * cct, axis=0)
        m = smax - ssum * np.float32(1.0 / L_K)
        rows.append(m.reshape(1, CH))
    M = jnp.concatenate(rows, axis=0) if n_ch > 1 else rows[0]

    flat = (jax.lax.broadcasted_iota(jnp.int32, (n_ch, CH), 0) * CH
            + jax.lax.broadcasted_iota(jnp.int32, (n_ch, CH), 1))
    sel = jnp.full((u_pad, 1), np.int32(2 ** 30), jnp.int32)
    sel_rows = jax.lax.broadcasted_iota(jnp.int32, (u_pad, 1), 0)
    Mw = M
    for i in range(u):
        mx = jnp.max(Mw)
        idx_i = jnp.min(jnp.where(Mw == mx, flat, np.int32(2 ** 30)))
        sel = jnp.where(sel_rows == i, idx_i, sel)
        Mw = jnp.where(flat == idx_i, _NEG, Mw)

    liota = jax.lax.broadcasted_iota(jnp.int32, (u_pad, L_Q), 1)
    R = (sel == liota).astype(jnp.float32)

    Qr = jax.lax.dot_general(R, Q, (((1,), (0,)), ((), ())),
                             preferred_element_type=jnp.float32)
    scores = jax.lax.dot_general(Qr, K, (((1,), (1,)), ((), ())),
                                 preferred_element_type=jnp.float32)
    scores = scores * np.float32(scale)
    smax2 = jnp.max(scores, axis=1, keepdims=True)
    e = jnp.exp(scores - smax2)
    attn = e / jnp.sum(e, axis=1, keepdims=True)
    upd = jax.lax.dot_general(attn, V, (((1,), (0,)), ((), ())),
                              preferred_element_type=jnp.float32)
    mean_v = jnp.sum(V, axis=0, keepdims=True) * np.float32(1.0 / L_K)
    delta = upd - mean_v
    ctx = jax.lax.dot_general(R, delta, (((0,), (0,)), ((), ())),
                              preferred_element_type=jnp.float32)
    o_ref[0] = ctx + mean_v


def kernel(queries, keys, values):
    B, L_Q, H, D = queries.shape
    L_K = keys.shape[1]
    U_part = min(_FACTOR * int(np.ceil(np.log(L_K))), L_K)
    u = min(_FACTOR * int(np.ceil(np.log(L_Q))), L_Q)
    u_pad = -(-u // 8) * 8
    scale = 1.0 / math.sqrt(D)
    ct = _sample_count_matrix_T(L_Q, L_K, U_part)

    q = jnp.transpose(queries, (0, 2, 1, 3)).reshape(B * H, L_Q, D)
    k = jnp.transpose(keys, (0, 2, 1, 3)).reshape(B * H, L_K, D)
    v = jnp.transpose(values, (0, 2, 1, 3)).reshape(B * H, L_K, D)

    out = pl.pallas_call(
        functools.partial(_body, u, u_pad, L_K, scale),
        grid=(B * H,),
        in_specs=[
            pl.BlockSpec((1, L_Q, D), lambda i: (i, 0, 0)),
            pl.BlockSpec((1, L_K, D), lambda i: (i, 0, 0)),
            pl.BlockSpec((1, L_K, D), lambda i: (i, 0, 0)),
            pl.BlockSpec((L_K, L_Q), lambda i: (0, 0)),
        ],
        out_specs=pl.BlockSpec((1, L_Q, D), lambda i: (i, 0, 0)),
        out_shape=jax.ShapeDtypeStruct((B * H, L_Q, D), jnp.float32),
        compiler_params=pltpu.CompilerParams(
            dimension_semantics=("arbitrary",),
        ),
    )(q, k, v, ct)

    return jnp.transpose(out.reshape(B, H, L_Q, D), (0, 2, 1, 3))


# speedup vs baseline: 2.8515x; 1.1540x over previous
import functools
import math

import numpy as np
import jax
import jax.numpy as jnp
from jax.experimental import pallas as pl
from jax.experimental.pallas import tpu as pltpu

_FACTOR = 5
_NEG = np.float32(-3.4e38)


def _threefry2x32_np(k1, k2, x0, x1):
    rot = (13, 15, 26, 6, 17, 29, 16, 24)

    def rotl(x, d):
        return (x << np.uint32(d)) | (x >> np.uint32(32 - d))

    ks = [np.uint32(k1), np.uint32(k2),
          np.uint32(np.uint32(k1) ^ np.uint32(k2) ^ np.uint32(0x1BD11BDA))]
    x = [x0 + ks[0], x1 + ks[1]]
    ksched = [(ks[1], ks[2]), (ks[2], ks[0]), (ks[0], ks[1]),
              (ks[1], ks[2]), (ks[2], ks[0])]
    for i in range(5):
        for r in rot[:4] if i % 2 == 0 else rot[4:]:
            x[0] = x[0] + x[1]
            x[1] = rotl(x[1], r)
            x[1] = x[0] ^ x[1]
        a, b = ksched[i]
        x[0] = x[0] + a
        x[1] = x[1] + b + np.uint32(i + 1)
    return x[0], x[1]


def _fixed_sample_indices(L_Q: int, L_K: int, U_part: int) -> np.ndarray:
    with np.errstate(over="ignore"):
        b1, b2 = _threefry2x32_np(np.uint32(0), np.uint32(42),
                                  np.zeros(2, np.uint32),
                                  np.arange(2, dtype=np.uint32))
        n = L_Q * U_part
        hi_cnt = np.zeros(n, np.uint32)
        lo_cnt = np.arange(n, dtype=np.uint32)
        ha, hb = _threefry2x32_np(b1[0], b2[0], hi_cnt, lo_cnt)
        la, lb = _threefry2x32_np(b1[1], b2[1], hi_cnt, lo_cnt)
        higher_bits, lower_bits = ha ^ hb, la ^ lb
        span = np.uint32(L_K)
        mult = np.uint32((2 ** 16) % L_K)
        mult = np.uint32((int(mult) * int(mult)) % L_K)
        offset = ((higher_bits % span) * mult + lower_bits % span) % span
    return offset.astype(np.int64).reshape(L_Q, U_part)


@functools.lru_cache(maxsize=None)
def _sample_count_matrix_T(L_Q: int, L_K: int, U_part: int) -> np.ndarray:
    idx_np = _fixed_sample_indices(L_Q, L_K, U_part)
    ct = np.zeros((L_K, L_Q), np.float32)
    np.add.at(ct, (idx_np.reshape(-1), np.repeat(np.arange(L_Q), U_part)), 1.0)
    return ct


_sample_count_matrix_T(2048, 2048, 40)


def _body(u, u_pad, L_K, scale, q_ref, k_ref, v_ref, ct_ref, o_ref):
    L_Q, D = q_ref.shape[1], q_ref.shape[2]
    Q = q_ref[0]
    K = k_ref[0]
    V = v_ref[0]

    CH = min(512, L_Q)
    n_ch = L_Q // CH
    rows = []
    for ci in range(n_ch):
        qc = Q[ci * CH:(ci + 1) * CH, :]
        qkt = jax.lax.dot_general(K, qc, (((1,), (1,)), ((), ())),
                                  preferred_element_type=jnp.float32)
        cct = ct_ref[:, ci * CH:(ci + 1) * CH]
        smax = jnp.max(jnp.where(cct > 0, qkt, _NEG), axis=0)
        ssum = jnp.sum(qkt * cct, axis=0)
        m = smax - ssum * np.float32(1.0 / L_K)
        rows.append(m.reshape(1, CH))
    M = jnp.concatenate(rows, axis=0) if n_ch > 1 else rows[0]

    flat = (jax.lax.broadcasted_iota(jnp.int32, (n_ch, CH), 0) * CH
            + jax.lax.broadcasted_iota(jnp.int32, (n_ch, CH), 1))
    sel = jnp.full((u_pad, 1), np.int32(2 ** 30), jnp.int32)
    sel_rows = jax.lax.broadcasted_iota(jnp.int32, (u_pad, 1), 0)
    Mw = M
    big = np.int32(2 ** 30)
    for i in range(u):
        mx = jnp.max(Mw, axis=1, keepdims=True)
        mx = jnp.max(mx, axis=0, keepdims=True)
        idx_c = jnp.where(Mw == mx, flat, big)
        idx_i = jnp.min(idx_c, axis=1, keepdims=True)
        idx_i = jnp.min(idx_i, axis=0, keepdims=True)
        sel = jnp.where(sel_rows == i, idx_i, sel)
        Mw = jnp.where(flat == idx_i, _NEG, Mw)

    liota = jax.lax.broadcasted_iota(jnp.int32, (u_pad, L_Q), 1)
    R = (sel == liota).astype(jnp.float32)

    Qr = jax.lax.dot_general(R, Q, (((1,), (0,)), ((), ())),
                             preferred_element_type=jnp.float32)
    scores = jax.lax.dot_general(Qr, K, (((1,), (1,)), ((), ())),
                                 preferred_element_type=jnp.float32)
    scores = scores * np.float32(scale)
    smax2 = jnp.max(scores, axis=1, keepdims=True)
    e = jnp.exp(scores - smax2)
    attn = e / jnp.sum(e, axis=1, keepdims=True)
    upd = jax.lax.dot_general(attn, V, (((1,), (0,)), ((), ())),
                              preferred_element_type=jnp.float32)
    mean_v = jnp.sum(V, axis=0, keepdims=True) * np.float32(1.0 / L_K)
    delta = upd - mean_v
    ctx = jax.lax.dot_general(R, delta, (((0,), (0,)), ((), ())),
                              preferred_element_type=jnp.float32)
    o_ref[0] = ctx + mean_v


def kernel(queries, keys, values):
    B, L_Q, H, D = queries.shape
    L_K = keys.shape[1]
    U_part = min(_FACTOR * int(np.ceil(np.log(L_K))), L_K)
    u = min(_FACTOR * int(np.ceil(np.log(L_Q))), L_Q)
    u_pad = -(-u // 8) * 8
    scale = 1.0 / math.sqrt(D)
    ct = _sample_count_matrix_T(L_Q, L_K, U_part)

    q = jnp.transpose(queries, (0, 2, 1, 3)).reshape(B * H, L_Q, D)
    k = jnp.transpose(keys, (0, 2, 1, 3)).reshape(B * H, L_K, D)
    v = jnp.transpose(values, (0, 2, 1, 3)).reshape(B * H, L_K, D)

    out = pl.pallas_call(
        functools.partial(_body, u, u_pad, L_K, scale),
        grid=(B * H,),
        in_specs=[
            pl.BlockSpec((1, L_Q, D), lambda i: (i, 0, 0)),
            pl.BlockSpec((1, L_K, D), lambda i: (i, 0, 0)),
            pl.BlockSpec((1, L_K, D), lambda i: (i, 0, 0)),
            pl.BlockSpec((L_K, L_Q), lambda i: (0, 0)),
        ],
        out_specs=pl.BlockSpec((1, L_Q, D), lambda i: (i, 0, 0)),
        out_shape=jax.ShapeDtypeStruct((B * H, L_Q, D), jnp.float32),
        compiler_params=pltpu.CompilerParams(
            dimension_semantics=("arbitrary",),
        ),
    )(q, k, v, ct)

    return jnp.transpose(out.reshape(B, H, L_Q, D), (0, 2, 1, 3))


# speedup vs baseline: 5.1187x; 1.7951x over previous
import functools
import math

import numpy as np
import jax
import jax.numpy as jnp
from jax.experimental import pallas as pl
from jax.experimental.pallas import tpu as pltpu

_FACTOR = 5
_NEG = np.float32(-3.4e38)


def _threefry2x32_np(k1, k2, x0, x1):
    rot = (13, 15, 26, 6, 17, 29, 16, 24)

    def rotl(x, d):
        return (x << np.uint32(d)) | (x >> np.uint32(32 - d))

    ks = [np.uint32(k1), np.uint32(k2),
          np.uint32(np.uint32(k1) ^ np.uint32(k2) ^ np.uint32(0x1BD11BDA))]
    x = [x0 + ks[0], x1 + ks[1]]
    ksched = [(ks[1], ks[2]), (ks[2], ks[0]), (ks[0], ks[1]),
              (ks[1], ks[2]), (ks[2], ks[0])]
    for i in range(5):
        for r in rot[:4] if i % 2 == 0 else rot[4:]:
            x[0] = x[0] + x[1]
            x[1] = rotl(x[1], r)
            x[1] = x[0] ^ x[1]
        a, b = ksched[i]
        x[0] = x[0] + a
        x[1] = x[1] + b + np.uint32(i + 1)
    return x[0], x[1]


def _fixed_sample_indices(L_Q: int, L_K: int, U_part: int) -> np.ndarray:
    with np.errstate(over="ignore"):
        b1, b2 = _threefry2x32_np(np.uint32(0), np.uint32(42),
                                  np.zeros(2, np.uint32),
                                  np.arange(2, dtype=np.uint32))
        n = L_Q * U_part
        hi_cnt = np.zeros(n, np.uint32)
        lo_cnt = np.arange(n, dtype=np.uint32)
        ha, hb = _threefry2x32_np(b1[0], b2[0], hi_cnt, lo_cnt)
        la, lb = _threefry2x32_np(b1[1], b2[1], hi_cnt, lo_cnt)
        higher_bits, lower_bits = ha ^ hb, la ^ lb
        span = np.uint32(L_K)
        mult = np.uint32((2 ** 16) % L_K)
        mult = np.uint32((int(mult) * int(mult)) % L_K)
        offset = ((higher_bits % span) * mult + lower_bits % span) % span
    return offset.astype(np.int64).reshape(L_Q, U_part)


@functools.lru_cache(maxsize=None)
def _sample_count_matrix_T(L_Q: int, L_K: int, U_part: int) -> np.ndarray:
    idx_np = _fixed_sample_indices(L_Q, L_K, U_part)
    ct = np.zeros((L_K, L_Q), np.float32)
    np.add.at(ct, (idx_np.reshape(-1), np.repeat(np.arange(L_Q), U_part)), 1.0)
    return ct


_sample_count_matrix_T(2048, 2048, 40)


def _body(u, u_pad, L_K, scale, q_ref, k_ref, v_ref, ct_ref, o_ref):
    L_Q, D = q_ref.shape[1], q_ref.shape[2]
    Q = q_ref[0]
    K = k_ref[0]
    V = v_ref[0]

    CH = min(512, L_Q)
    n_ch = L_Q // CH
    rows = []
    for ci in range(n_ch):
        qc = Q[ci * CH:(ci + 1) * CH, :]
        qkt = jax.lax.dot_general(K, qc, (((1,), (1,)), ((), ())),
                                  preferred_element_type=jnp.float32)
        cct = ct_ref[:, ci * CH:(ci + 1) * CH]
        smax = jnp.max(jnp.where(cct > 0, qkt, _NEG), axis=0)
        ssum = jnp.sum(qkt * cct, axis=0)
        m = smax - ssum * np.float32(1.0 / L_K)
        rows.append(m.reshape(1, CH))
    M = jnp.concatenate(rows, axis=1) if n_ch > 1 else rows[0]

    lane_idx = jax.lax.broadcasted_iota(jnp.int32, (1, L_Q), 1)
    rank = jnp.zeros((1, L_Q), jnp.float32)
    RC = min(512, L_Q)
    eye = (jax.lax.broadcasted_iota(jnp.int32, (RC, RC), 0)
           == jax.lax.broadcasted_iota(jnp.int32, (RC, RC), 1)
           ).astype(jnp.float32)
    for rc in range(L_Q // RC):
        mrow = jax.lax.dot_general(
            eye, M[:, rc * RC:(rc + 1) * RC], (((1,), (1,)), ((), ())),
            preferred_element_type=jnp.float32)
        rowi = (jax.lax.broadcasted_iota(jnp.int32, (RC, 1), 0)
                + np.int32(rc * RC))
        gt = mrow > M
        eq = mrow == M
        tri = rowi < lane_idx
        cond = jnp.logical_or(gt, jnp.logical_and(eq, tri))
        condf = jnp.where(cond, np.float32(1.0), np.float32(0.0))
        rank = rank + jnp.sum(condf, axis=0, keepdims=True)

    ranki = rank.astype(jnp.int32)
    uio = jax.lax.broadcasted_iota(jnp.int32, (u_pad, 1), 0)
    R = ((ranki == uio) & (uio < np.int32(u))).astype(jnp.float32)

    Qr = jax.lax.dot_general(R, Q, (((1,), (0,)), ((), ())),
                             preferred_element_type=jnp.float32)
    scores = jax.lax.dot_general(Qr, K, (((1,), (1,)), ((), ())),
                                 preferred_element_type=jnp.float32)
    scores = scores * np.float32(scale)
    smax2 = jnp.max(scores, axis=1, keepdims=True)
    e = jnp.exp(scores - smax2)
    attn = e / jnp.sum(e, axis=1, keepdims=True)
    upd = jax.lax.dot_general(attn, V, (((1,), (0,)), ((), ())),
                              preferred_element_type=jnp.float32)
    mean_v = jnp.sum(V, axis=0, keepdims=True) * np.float32(1.0 / L_K)
    delta = upd - mean_v
    ctx = jax.lax.dot_general(R, delta, (((0,), (0,)), ((), ())),
                              preferred_element_type=jnp.float32)
    o_ref[0] = ctx + mean_v


def kernel(queries, keys, values):
    B, L_Q, H, D = queries.shape
    L_K = keys.shape[1]
    U_part = min(_FACTOR * int(np.ceil(np.log(L_K))), L_K)
    u = min(_FACTOR * int(np.ceil(np.log(L_Q))), L_Q)
    u_pad = -(-u // 8) * 8
    scale = 1.0 / math.sqrt(D)
    ct = _sample_count_matrix_T(L_Q, L_K, U_part)

    q = jnp.transpose(queries, (0, 2, 1, 3)).reshape(B * H, L_Q, D)
    k = jnp.transpose(keys, (0, 2, 1, 3)).reshape(B * H, L_K, D)
    v = jnp.transpose(values, (0, 2, 1, 3)).reshape(B * H, L_K, D)

    out = pl.pallas_call(
        functools.partial(_body, u, u_pad, L_K, scale),
        grid=(B * H,),
        in_specs=[
            pl.BlockSpec((1, L_Q, D), lambda i: (i, 0, 0)),
            pl.BlockSpec((1, L_K, D), lambda i: (i, 0, 0)),
            pl.BlockSpec((1, L_K, D), lambda i: (i, 0, 0)),
            pl.BlockSpec((L_K, L_Q), lambda i: (0, 0)),
        ],
        out_specs=pl.BlockSpec((1, L_Q, D), lambda i: (i, 0, 0)),
        out_shape=jax.ShapeDtypeStruct((B * H, L_Q, D), jnp.float32),
        compiler_params=pltpu.CompilerParams(
            dimension_semantics=("arbitrary",),
        ),
    )(q, k, v, ct)

    return jnp.transpose(out.reshape(B, H, L_Q, D), (0, 2, 1, 3))


# speedup vs baseline: 5.2024x; 1.0163x over previous
import functools
import math

import numpy as np
import jax
import jax.numpy as jnp
from jax.experimental import pallas as pl
from jax.experimental.pallas import tpu as pltpu

_FACTOR = 5
_NEG = np.float32(-3.4e38)


def _threefry2x32_np(k1, k2, x0, x1):
    rot = (13, 15, 26, 6, 17, 29, 16, 24)

    def rotl(x, d):
        return (x << np.uint32(d)) | (x >> np.uint32(32 - d))

    ks = [np.uint32(k1), np.uint32(k2),
          np.uint32(np.uint32(k1) ^ np.uint32(k2) ^ np.uint32(0x1BD11BDA))]
    x = [x0 + ks[0], x1 + ks[1]]
    ksched = [(ks[1], ks[2]), (ks[2], ks[0]), (ks[0], ks[1]),
              (ks[1], ks[2]), (ks[2], ks[0])]
    for i in range(5):
        for r in rot[:4] if i % 2 == 0 else rot[4:]:
            x[0] = x[0] + x[1]
            x[1] = rotl(x[1], r)
            x[1] = x[0] ^ x[1]
        a, b = ksched[i]
        x[0] = x[0] + a
        x[1] = x[1] + b + np.uint32(i + 1)
    return x[0], x[1]


def _fixed_sample_indices(L_Q: int, L_K: int, U_part: int) -> np.ndarray:
    with np.errstate(over="ignore"):
        b1, b2 = _threefry2x32_np(np.uint32(0), np.uint32(42),
                                  np.zeros(2, np.uint32),
                                  np.arange(2, dtype=np.uint32))
        n = L_Q * U_part
        hi_cnt = np.zeros(n, np.uint32)
        lo_cnt = np.arange(n, dtype=np.uint32)
        ha, hb = _threefry2x32_np(b1[0], b2[0], hi_cnt, lo_cnt)
        la, lb = _threefry2x32_np(b1[1], b2[1], hi_cnt, lo_cnt)
        higher_bits, lower_bits = ha ^ hb, la ^ lb
        span = np.uint32(L_K)
        mult = np.uint32((2 ** 16) % L_K)
        mult = np.uint32((int(mult) * int(mult)) % L_K)
        offset = ((higher_bits % span) * mult + lower_bits % span) % span
    return offset.astype(np.int64).reshape(L_Q, U_part)


@functools.lru_cache(maxsize=None)
def _sample_count_matrix_T(L_Q: int, L_K: int, U_part: int):
    idx_np = _fixed_sample_indices(L_Q, L_K, U_part)
    ct = np.zeros((L_K, L_Q), np.float32)
    np.add.at(ct, (idx_np.reshape(-1), np.repeat(np.arange(L_Q), U_part)), 1.0)
    madd = np.where(ct > 0, np.float32(0.0), _NEG).astype(np.float32)
    return ct, madd


_sample_count_matrix_T(2048, 2048, 40)


def _body(u, u_pad, L_K, scale, q_ref, k_ref, v_ref, ct_ref, madd_ref, o_ref):
    L_Q, D = q_ref.shape[1], q_ref.shape[2]
    Q = q_ref[0]
    K = k_ref[0]
    V = v_ref[0]

    CH = min(512, L_Q)
    n_ch = L_Q // CH
    ones8k = jnp.full((8, L_K), np.float32(1.0), jnp.float32)
    rows = []
    for ci in range(n_ch):
        qc = Q[ci * CH:(ci + 1) * CH, :]
        qkt = jax.lax.dot_general(K, qc, (((1,), (1,)), ((), ())),
                                  preferred_element_type=jnp.float32)
        cct = ct_ref[:, ci * CH:(ci + 1) * CH]
        ma = madd_ref[:, ci * CH:(ci + 1) * CH]
        smax = jnp.max(qkt + ma, axis=0)
        ssum = jnp.sum(qkt * cct, axis=0)
        m = smax - ssum * np.float32(1.0 / L_K)
        rows.append(m.reshape(1, CH))
    M = jnp.concatenate(rows, axis=1) if n_ch > 1 else rows[0]

    lane_idx = jax.lax.broadcasted_iota(jnp.int32, (1, L_Q), 1)
    rank = jnp.zeros((1, L_Q), jnp.float32)
    RC = min(512, L_Q)
    ones8r = jnp.full((8, RC), np.float32(1.0), jnp.float32)
    eye = (jax.lax.broadcasted_iota(jnp.int32, (RC, RC), 0)
           == jax.lax.broadcasted_iota(jnp.int32, (RC, RC), 1)
           ).astype(jnp.float32)
    for rc in range(L_Q // RC):
        mrow = jax.lax.dot_general(
            eye, M[:, rc * RC:(rc + 1) * RC], (((1,), (1,)), ((), ())),
            preferred_element_type=jnp.float32)
        rowi = (jax.lax.broadcasted_iota(jnp.int32, (RC, 1), 0)
                + np.int32(rc * RC))
        gt = mrow > M
        eq = mrow == M
        tri = rowi < lane_idx
        cond = jnp.logical_or(gt, jnp.logical_and(eq, tri))
        condf = jnp.where(cond, np.float32(1.0), np.float32(0.0))
        rank = rank + jnp.sum(condf, axis=0, keepdims=True)

    ranki = rank.astype(jnp.int32)
    uio = jax.lax.broadcasted_iota(jnp.int32, (u_pad, 1), 0)
    R = ((ranki == uio) & (uio < np.int32(u))).astype(jnp.float32)

    Qr = jax.lax.dot_general(R, Q, (((1,), (0,)), ((), ())),
                             preferred_element_type=jnp.float32)
    scores = jax.lax.dot_general(Qr, K, (((1,), (1,)), ((), ())),
                                 preferred_element_type=jnp.float32)
    scores = scores * np.float32(scale)
    smax2 = jnp.max(scores, axis=1, keepdims=True)
    e = jnp.exp(scores - smax2)
    attn = e / jnp.sum(e, axis=1, keepdims=True)
    upd = jax.lax.dot_general(attn, V, (((1,), (0,)), ((), ())),
                              preferred_element_type=jnp.float32)
    mean_v = jnp.sum(V, axis=0, keepdims=True) * np.float32(1.0 / L_K)
    delta = upd - mean_v
    ctx = jax.lax.dot_general(R, delta, (((0,), (0,)), ((), ())),
                              preferred_element_type=jnp.float32)
    o_ref[0] = ctx + mean_v


def kernel(queries, keys, values):
    B, L_Q, H, D = queries.shape
    L_K = keys.shape[1]
    U_part = min(_FACTOR * int(np.ceil(np.log(L_K))), L_K)
    u = min(_FACTOR * int(np.ceil(np.log(L_Q))), L_Q)
    u_pad = -(-u // 8) * 8
    scale = 1.0 / math.sqrt(D)
    ct, madd = _sample_count_matrix_T(L_Q, L_K, U_part)

    q = jnp.transpose(queries, (0, 2, 1, 3)).reshape(B * H, L_Q, D)
    k = jnp.transpose(keys, (0, 2, 1, 3)).reshape(B * H, L_K, D)
    v = jnp.transpose(values, (0, 2, 1, 3)).reshape(B * H, L_K, D)

    out = pl.pallas_call(
        functools.partial(_body, u, u_pad, L_K, scale),
        grid=(B * H,),
        in_specs=[
            pl.BlockSpec((1, L_Q, D), lambda i: (i, 0, 0)),
            pl.BlockSpec((1, L_K, D), lambda i: (i, 0, 0)),
            pl.BlockSpec((1, L_K, D), lambda i: (i, 0, 0)),
            pl.BlockSpec((L_K, L_Q), lambda i: (0, 0)),
            pl.BlockSpec((L_K, L_Q), lambda i: (0, 0)),
        ],
        out_specs=pl.BlockSpec((1, L_Q, D), lambda i: (i, 0, 0)),
        out_shape=jax.ShapeDtypeStruct((B * H, L_Q, D), jnp.float32),
        compiler_params=pltpu.CompilerParams(
            dimension_semantics=("arbitrary",),
        ),
    )(q, k, v, ct, madd)

    return jnp.transpose(out.reshape(B, H, L_Q, D), (0, 2, 1, 3))


# speedup vs baseline: 5.5139x; 1.0599x over previous
import functools
import math

import numpy as np
import jax
import jax.numpy as jnp
from jax.experimental import pallas as pl
from jax.experimental.pallas import tpu as pltpu

_FACTOR = 5
_NEG = np.float32(-3.4e38)


def _threefry2x32_np(k1, k2, x0, x1):
    rot = (13, 15, 26, 6, 17, 29, 16, 24)

    def rotl(x, d):
        return (x << np.uint32(d)) | (x >> np.uint32(32 - d))

    ks = [np.uint32(k1), np.uint32(k2),
          np.uint32(np.uint32(k1) ^ np.uint32(k2) ^ np.uint32(0x1BD11BDA))]
    x = [x0 + ks[0], x1 + ks[1]]
    ksched = [(ks[1], ks[2]), (ks[2], ks[0]), (ks[0], ks[1]),
              (ks[1], ks[2]), (ks[2], ks[0])]
    for i in range(5):
        for r in rot[:4] if i % 2 == 0 else rot[4:]:
            x[0] = x[0] + x[1]
            x[1] = rotl(x[1], r)
            x[1] = x[0] ^ x[1]
        a, b = ksched[i]
        x[0] = x[0] + a
        x[1] = x[1] + b + np.uint32(i + 1)
    return x[0], x[1]


def _fixed_sample_indices(L_Q: int, L_K: int, U_part: int) -> np.ndarray:
    with np.errstate(over="ignore"):
        b1, b2 = _threefry2x32_np(np.uint32(0), np.uint32(42),
                                  np.zeros(2, np.uint32),
                                  np.arange(2, dtype=np.uint32))
        n = L_Q * U_part
        hi_cnt = np.zeros(n, np.uint32)
        lo_cnt = np.arange(n, dtype=np.uint32)
        ha, hb = _threefry2x32_np(b1[0], b2[0], hi_cnt, lo_cnt)
        la, lb = _threefry2x32_np(b1[1], b2[1], hi_cnt, lo_cnt)
        higher_bits, lower_bits = ha ^ hb, la ^ lb
        span = np.uint32(L_K)
        mult = np.uint32((2 ** 16) % L_K)
        mult = np.uint32((int(mult) * int(mult)) % L_K)
        offset = ((higher_bits % span) * mult + lower_bits % span) % span
    return offset.astype(np.int64).reshape(L_Q, U_part)


@functools.lru_cache(maxsize=None)
def _sample_count_matrix_T(L_Q: int, L_K: int, U_part: int):
    idx_np = _fixed_sample_indices(L_Q, L_K, U_part)
    ct = np.zeros((L_K, L_Q), np.float32)
    np.add.at(ct, (idx_np.reshape(-1), np.repeat(np.arange(L_Q), U_part)), 1.0)
    madd = np.where(ct > 0, np.float32(0.0), _NEG).astype(np.float32)
    return ct, madd


_sample_count_matrix_T(2048, 2048, 40)


def _body(u, u_pad, L_K, D, scale, q_ref, k_ref, v_ref, ct_ref, madd_ref,
          o_ref):
    for h in range(q_ref.shape[2] // D):
        _one_head(u, u_pad, L_K, D, scale, h, q_ref, k_ref, v_ref, ct_ref,
                  madd_ref, o_ref)


def _one_head(u, u_pad, L_K, D, scale, h, q_ref, k_ref, v_ref, ct_ref,
              madd_ref, o_ref):
    L_Q = q_ref.shape[1]
    Q = q_ref[0, :, h * D:(h + 1) * D]
    K = k_ref[0, :, h * D:(h + 1) * D]
    V = v_ref[0, :, h * D:(h + 1) * D]

    CH = min(512, L_Q)
    n_ch = L_Q // CH
    ones8k = jnp.full((8, L_K), np.float32(1.0), jnp.float32)
    rows = []
    for ci in range(n_ch):
        qc = Q[ci * CH:(ci + 1) * CH, :]
        qkt = jax.lax.dot_general(K, qc, (((1,), (1,)), ((), ())),
                                  preferred_element_type=jnp.float32)
        cct = ct_ref[:, ci * CH:(ci + 1) * CH]
        ma = madd_ref[:, ci * CH:(ci + 1) * CH]
        smax = jnp.max(qkt + ma, axis=0)
        ssum = jnp.sum(qkt * cct, axis=0)
        m = smax - ssum * np.float32(1.0 / L_K)
        rows.append(m.reshape(1, CH))
    M = jnp.concatenate(rows, axis=1) if n_ch > 1 else rows[0]

    lane_idx = jax.lax.broadcasted_iota(jnp.int32, (1, L_Q), 1)
    rank = jnp.zeros((1, L_Q), jnp.float32)
    RC = min(512, L_Q)
    ones8r = jnp.full((8, RC), np.float32(1.0), jnp.float32)
    eye = (jax.lax.broadcasted_iota(jnp.int32, (RC, RC), 0)
           == jax.lax.broadcasted_iota(jnp.int32, (RC, RC), 1)
           ).astype(jnp.float32)
    for rc in range(L_Q // RC):
        mrow = jax.lax.dot_general(
            eye, M[:, rc * RC:(rc + 1) * RC], (((1,), (1,)), ((), ())),
            preferred_element_type=jnp.float32)
        rowi = (jax.lax.broadcasted_iota(jnp.int32, (RC, 1), 0)
                + np.int32(rc * RC))
        gt = mrow > M
        eq = mrow == M
        tri = rowi < lane_idx
        cond = jnp.logical_or(gt, jnp.logical_and(eq, tri))
        condf = jnp.where(cond, np.float32(1.0), np.float32(0.0))
        rank = rank + jnp.sum(condf, axis=0, keepdims=True)

    ranki = rank.astype(jnp.int32)
    uio = jax.lax.broadcasted_iota(jnp.int32, (u_pad, 1), 0)
    R = ((ranki == uio) & (uio < np.int32(u))).astype(jnp.float32)

    Qr = jax.lax.dot_general(R, Q, (((1,), (0,)), ((), ())),
                             preferred_element_type=jnp.float32)
    scores = jax.lax.dot_general(Qr, K, (((1,), (1,)), ((), ())),
                                 preferred_element_type=jnp.float32)
    scores = scores * np.float32(scale)
    smax2 = jnp.max(scores, axis=1, keepdims=True)
    e = jnp.exp(scores - smax2)
    attn = e / jnp.sum(e, axis=1, keepdims=True)
    upd = jax.lax.dot_general(attn, V, (((1,), (0,)), ((), ())),
                              preferred_element_type=jnp.float32)
    mean_v = jnp.sum(V, axis=0, keepdims=True) * np.float32(1.0 / L_K)
    delta = upd - mean_v
    ctx = jax.lax.dot_general(R, delta, (((0,), (0,)), ((), ())),
                              preferred_element_type=jnp.float32)
    o_ref[0, :, h * D:(h + 1) * D] = ctx + mean_v


def kernel(queries, keys, values):
    B, L_Q, H, D = queries.shape
    L_K = keys.shape[1]
    U_part = min(_FACTOR * int(np.ceil(np.log(L_K))), L_K)
    u = min(_FACTOR * int(np.ceil(np.log(L_Q))), L_Q)
    u_pad = -(-u // 8) * 8
    scale = 1.0 / math.sqrt(D)
    ct, madd = _sample_count_matrix_T(L_Q, L_K, U_part)

    HPB = max(1, 128 // D)
    ng = H // HPB
    q3 = queries.reshape(B, L_Q, H * D)
    k3 = keys.reshape(B, L_K, H * D)
    v3 = values.reshape(B, L_K, H * D)
    bh = pl.BlockSpec((1, L_Q, HPB * D), lambda i: (i // ng, 0, i % ng))
    out = pl.pallas_call(
        functools.partial(_body, u, u_pad, L_K, D, scale),
        grid=(B * ng,),
        in_specs=[
            bh,
            bh,
            bh,
            pl.BlockSpec((L_K, L_Q), lambda i: (0, 0)),
            pl.BlockSpec((L_K, L_Q), lambda i: (0, 0)),
        ],
        out_specs=bh,
        out_shape=jax.ShapeDtypeStruct((B, L_Q, H * D), jnp.float32),
        compiler_params=pltpu.CompilerParams(
            dimension_semantics=("arbitrary",),
        ),
    )(q3, k3, v3, ct, madd)

    return out.reshape(B, L_Q, H, D)
